# Initial kernel scaffold; baseline (speedup 1.0000x reference)
import jax
import jax.numpy as jnp
from jax.experimental import pallas as pl


def kernel(x, pos, edge_index, t, noise, We1, be1, We2, be2, Wc1, bc1, Wc2, bc2, Wn1, bn1, Wn2, bn2):
    raise NotImplementedError("write your pallas kernel here")


# speedup vs baseline: 3.1357x; 3.1357x over previous
import dataclasses
import functools

import jax
import jax.numpy as jnp
from jax import lax
from jax.experimental import pallas as pl
from jax.experimental.pallas import tpu as pltpu
from jax.experimental.pallas import tpu_sc as plsc

_NC = 2
_NS = 16
_NW = _NC * _NS
_CH = 128
_PREC = jax.lax.Precision.HIGHEST


def _silu(v):
    return v * (1.0 / (1.0 + jnp.exp(-v)))


def _mesh():
    return plsc.VectorSubcoreMesh(core_axis_name="c", subcore_axis_name="s")


def _sc_params():
    cp = pltpu.CompilerParams()
    if "needs_layout_passes" in pltpu.CompilerParams.__dataclass_fields__:
        cp = dataclasses.replace(cp, needs_layout_passes=False)
    return cp


def _dot(a, b):
    return jnp.dot(a.astype(jnp.bfloat16), b.astype(jnp.bfloat16),
                   preferred_element_type=jnp.float32)


def _b(v):
    return v.astype(jnp.bfloat16).astype(jnp.float32)


def _sc_gather(A, B, p4flat, src, dst):
    N, D = A.shape
    E = src.shape[0]
    nch = E // _CH
    npass = (nch + _NW - 1) // _NW

    @functools.partial(
        pl.kernel,
        out_type=(jax.ShapeDtypeStruct((E, D), jnp.float32),
                  jax.ShapeDtypeStruct((E * 4,), jnp.float32)),
        mesh=_mesh(),
        scratch_types=[pltpu.VMEM((_CH,), jnp.int32),
                       pltpu.VMEM((_CH,), jnp.int32),
                       pltpu.VMEM((_CH, D), jnp.float32),
                       pltpu.VMEM((_CH, D), jnp.float32),
                       pltpu.VMEM((N * 4,), jnp.float32),
                       pltpu.VMEM((_CH * 4,), jnp.float32),
                       pltpu.SemaphoreType.DMA],
        compiler_params=_sc_params(),
    )
    def k(a_hbm, b_hbm, p4_hbm, src_hbm, dst_hbm, pre_hbm, rel_hbm,
          sv, dv, ba, bb, p4v, relv, sem):
        w = lax.axis_index("c") * _NS + lax.axis_index("s")
        pltpu.sync_copy(p4_hbm, p4v)
        zero16 = jnp.zeros((16,), jnp.float32)

        @pl.loop(0, _CH * 4, step=16)
        def _(i):
            relv[pl.ds(i, 16)] = zero16

        iota = lax.iota(jnp.int32, 16)

        @pl.loop(0, npass)
        def _(i):
            g = i * _NW + w

            @pl.when(g < nch)
            def _():
                off = g * _CH
                pltpu.sync_copy(src_hbm.at[pl.ds(off, _CH)], sv)
                pltpu.sync_copy(dst_hbm.at[pl.ds(off, _CH)], dv)
                c1 = pltpu.async_copy(a_hbm.at[dv], ba, sem)
                c2 = pltpu.async_copy(b_hbm.at[sv], bb, sem)

                @pl.loop(0, _CH // 16)
                def _(j):
                    d16 = dv[pl.ds(j * 16, 16)] * 4
                    s16 = sv[pl.ds(j * 16, 16)] * 4
                    out_ix = j * 64 + iota * 4
                    for c in range(3):
                        pd = plsc.load_gather(p4v, [d16 + c])
                        ps = plsc.load_gather(p4v, [s16 + c])
                        plsc.store_scatter(relv, [out_ix + c], pd - ps)

                c1.wait()
                c2.wait()

                @pl.loop(0, _CH)
                def _(r):
                    for cb in range(D // 16):
                        s_ = pl.ds(cb * 16, 16)
                        ba[r, s_] = ba[r, s_] + bb[r, s_]

                pltpu.sync_copy(ba, pre_hbm.at[pl.ds(off, _CH)])
                pltpu.sync_copy(relv, rel_hbm.at[pl.ds(off * 4, _CH * 4)])

    return k(A, B, p4flat, src, dst)


def _sc_scatter_m(m, dst, z128):
    E, D = m.shape
    N = z128.shape[0]
    nch = E // _CH
    npass = (nch + _NW - 1) // _NW
    rpt = (N // _NS) // 8 * 8
    tail = N - rpt * _NS

    @functools.partial(
        pl.kernel,
        out_type=jax.ShapeDtypeStruct((_NC * N, D), jnp.float32),
        mesh=_mesh(),
        scratch_types=[pltpu.VMEM((_CH,), jnp.int32),
                       pltpu.VMEM((_CH, D), jnp.float32),
                       pltpu.VMEM_SHARED((N, D), jnp.float32),
                       pltpu.SemaphoreType.DMA],
        compiler_params=_sc_params(),
    )
    def k(m_hbm, dst_hbm, z128_hbm, am_hbm, iv, mv, shm, sem):
        c = lax.axis_index("c")
        s = lax.axis_index("s")
        w_ = c * _NS + s
        r0 = s * rpt
        pltpu.sync_copy(z128_hbm.at[pl.ds(r0, rpt)], shm.at[pl.ds(r0, rpt)])

        @pl.when(s == 0)
        def _():
            pltpu.sync_copy(z128_hbm.at[pl.ds(rpt * _NS, tail)],
                            shm.at[pl.ds(rpt * _NS, tail)])

        plsc.subcore_barrier()

        @pl.loop(0, npass)
        def _(i):
            g = i * _NW + w_

            @pl.when(g < nch)
            def _():
                off = g * _CH
                pltpu.sync_copy(dst_hbm.at[pl.ds(off, _CH)], iv)
                pltpu.sync_copy(m_hbm.at[pl.ds(off, _CH)], mv)
                pltpu.sync_copy(mv, shm.at[iv], add=True)

        plsc.subcore_barrier()
        pltpu.sync_copy(shm.at[pl.ds(r0, rpt)],
                        am_hbm.at[pl.ds(c * N + r0, rpt)])

        @pl.when(s == 0)
        def _():
            pltpu.sync_copy(shm.at[pl.ds(rpt * _NS, tail)],
                            am_hbm.at[pl.ds(c * N + rpt * _NS, tail)])

    return k(m, dst, z128)


def _sc_scatter_pos(wrel4flat, dst, z4flat, N):
    E4 = wrel4flat.shape[0]
    E = E4 // 4
    nch = E // _CH
    npass = (nch + _NW - 1) // _NW

    @functools.partial(
        pl.kernel,
        out_type=jax.ShapeDtypeStruct((_NW * N * 4,), jnp.float32),
        mesh=_mesh(),
        scratch_types=[pltpu.VMEM((_CH,), jnp.int32),
                       pltpu.VMEM((_CH * 4,), jnp.float32),
                       pltpu.VMEM((N * 4,), jnp.float32),
                       pltpu.SemaphoreType.DMA],
        compiler_params=_sc_params(),
    )
    def k(w_hbm, dst_hbm, z4_hbm, ap_hbm, iv, wv, pacc, sem):
        c = lax.axis_index("c")
        s = lax.axis_index("s")
        w_ = c * _NS + s
        pltpu.sync_copy(z4_hbm, pacc)
        iota = lax.iota(jnp.int32, 16)

        @pl.loop(0, npass)
        def _(i):
            g = i * _NW + w_

            @pl.when(g < nch)
            def _():
                off = g * _CH
                pltpu.sync_copy(dst_hbm.at[pl.ds(off, _CH)], iv)
                pltpu.sync_copy(w_hbm.at[pl.ds(off * 4, _CH * 4)], wv)

                @pl.loop(0, _CH // 16)
                def _(j):
                    d16 = iv[pl.ds(j * 16, 16)] * 4
                    in_ix = j * 64 + iota * 4
                    for cc in range(4):
                        val = plsc.load_gather(wv, [in_ix + cc])
                        plsc.addupdate_scatter(pacc, [d16 + cc], val)

        pltpu.sync_copy(pacc, ap_hbm.at[pl.ds(w_ * (N * 4), N * 4)])

    return k(wrel4flat, dst, z4flat)


def _full(shape):
    return pl.BlockSpec(shape, lambda i: (0,) * len(shape))


def _tc_prep(pospad, noisepad, x, Wa, Wb, sab):
    N, D = x.shape
    BN = 2000

    def body(pp, np_, x_, wa, wb, sab_, p_o, a_o, b_o):
        p_o[...] = sab_[0, 0] * pp[...] + sab_[0, 1] * np_[...]
        a_o[...] = _dot(x_[...], wa[...])
        b_o[...] = _dot(x_[...], wb[...])

    row = pl.BlockSpec((BN, 4), lambda i: (i, 0))
    big = pl.BlockSpec((BN, D), lambda i: (i, 0))
    return pl.pallas_call(
        body,
        grid=(N // BN,),
        in_specs=[row, row, big, _full((D, D)), _full((D, D)), _full((1, 2))],
        out_specs=[row, big, big],
        out_shape=(jax.ShapeDtypeStruct((N, 4), jnp.float32),
                   jax.ShapeDtypeStruct((N, D), jnp.float32),
                   jax.ShapeDtypeStruct((N, D), jnp.float32)),
    )(pospad, noisepad, x, Wa, Wb, sab)


def _tc_edge(pre, rel, wc, b1, We2l, b2, Wc1l, bc1l, wc2r, bc2l, write_m):
    E, D = pre.shape
    BE = 2000

    def body(pre_ref, rel_ref, wc_ref, b1_ref, w2_ref, b2_ref, wc1_ref,
             bc1_ref, wc2_ref, bc2_ref, *outs):
        relv = rel_ref[...]
        d2 = jnp.sum(relv * relv, axis=1, keepdims=True)
        m1 = _silu(pre_ref[...] + _b(d2) * _b(wc_ref[...]) + b1_ref[...])
        m = _silu(_dot(m1, w2_ref[...]) + b2_ref[...])
        c1 = _silu(_dot(m, wc1_ref[...]) + bc1_ref[...])
        coef = (jnp.sum(_b(c1) * _b(wc2_ref[...]), axis=1, keepdims=True)
                + bc2_ref[0, 0])
        lane = lax.broadcasted_iota(jnp.int32, (BE, 4), 1)
        wrel = relv * coef + (lane == 3).astype(jnp.float32)
        if write_m:
            outs[0][...] = m
            outs[1][...] = wrel
        else:
            outs[0][...] = wrel

    big = pl.BlockSpec((BE, D), lambda i: (i, 0))
    row = pl.BlockSpec((BE, 4), lambda i: (i, 0))
    wspec = _full((D, D))
    vspec = _full((1, D))
    out_specs = [row]
    out_shape = [jax.ShapeDtypeStruct((E, 4), jnp.float32)]
    if write_m:
        out_specs = [big] + out_specs
        out_shape = [jax.ShapeDtypeStruct((E, D), jnp.float32)] + out_shape
    return pl.pallas_call(
        body,
        grid=(E // BE,),
        in_specs=[big, row, vspec, vspec, wspec, vspec, wspec, vspec, vspec,
                  _full((1, 1))],
        out_specs=out_specs,
        out_shape=tuple(out_shape),
    )(pre, rel, wc, b1, We2l, b2, Wc1l, bc1l, wc2r, bc2l)


def _tc_reduce_pos(ap2d):
    NW, F = ap2d.shape

    def body(ap_ref, o_ref):
        o_ref[...] = jnp.sum(ap_ref[...], axis=0)

    return pl.pallas_call(
        body,
        grid=(1,),
        in_specs=[pl.BlockSpec((NW, F), lambda i: (0, 0))],
        out_specs=pl.BlockSpec((F,), lambda i: (0,)),
        out_shape=jax.ShapeDtypeStruct((F,), jnp.float32),
    )(ap2d)


def _tc_node(h, P, am2, ap, Wn1a, Wn1b, b1, Wn2l, b2, Wa, Wb):
    N, D = h.shape
    BN = 2000
    nb = N // BN
    with_ab = Wa is not None

    def body(h_ref, p_ref, am0, am1, ap_ref, w1a, w1b, b1_ref, w2, b2_ref,
             *rest):
        if with_ab:
            wa, wb, h_o, p_o, a_o, b_o = rest
        else:
            h_o, p_o, a_o, b_o = rest[:2] + (None, None)
        aggm = am0[...] + am1[...]
        aggp = ap_ref[...]
        deg = jnp.maximum(aggp[:, 3:4], 1.0)
        p_o[...] = p_ref[...] + aggp / deg
        u = _silu(_dot(h_ref[...], w1a[...]) + _dot(aggm, w1b[...])
                  + b1_ref[...])
        hn = h_ref[...] + _dot(u, w2[...]) + b2_ref[...]
        h_o[...] = hn
        if with_ab:
            a_o[...] = _dot(hn, wa[...])
            b_o[...] = _dot(hn, wb[...])

    big = pl.BlockSpec((BN, D), lambda i: (i, 0))
    row = pl.BlockSpec((BN, 4), lambda i: (i, 0))
    big1 = pl.BlockSpec((BN, D), lambda i: (i + nb, 0))
    wspec = _full((D, D))
    vspec = _full((1, D))
    in_specs = [big, row, big, big1, row, wspec, wspec, vspec,
                wspec, vspec]
    args = [h, P, am2, am2, ap, Wn1a, Wn1b, b1, Wn2l, b2]
    out_specs = [big, row]
    out_shape = [jax.ShapeDtypeStruct((N, D), jnp.float32),
                 jax.ShapeDtypeStruct((N, 4), jnp.float32)]
    if with_ab:
        in_specs += [wspec, wspec]
        args += [Wa, Wb]
        out_specs += [big, big]
        out_shape += [jax.ShapeDtypeStruct((N, D), jnp.float32),
                      jax.ShapeDtypeStruct((N, D), jnp.float32)]
    return pl.pallas_call(
        body,
        grid=(nb,),
        in_specs=in_specs,
        out_specs=out_specs,
        out_shape=tuple(out_shape),
    )(*args)


def _tc_pos_update(P, ap):
    N = P.shape[0]
    BN = 2000
    nb = N // BN

    def body(p_ref, ap_ref, p_o):
        aggp = ap_ref[...]
        deg = jnp.maximum(aggp[:, 3:4], 1.0)
        p_o[...] = p_ref[...] + aggp / deg

    row = pl.BlockSpec((BN, 4), lambda i: (i, 0))
    return pl.pallas_call(
        body,
        grid=(nb,),
        in_specs=[row, row],
        out_specs=row,
        out_shape=jax.ShapeDtypeStruct((N, 4), jnp.float32),
    )(P, ap)


def kernel(x, pos, edge_index, t, noise, We1, be1, We2, be2, Wc1, bc1, Wc2,
           bc2, Wn1, bn1, Wn2, bn2):
    N, D = x.shape
    E = edge_index.shape[1]
    L = We1.shape[0]
    src = edge_index[0]
    dst = edge_index[1]

    beta_t = 0.0001 + (0.02 - 0.0001) * (jnp.float32(t) / 999.0)
    alpha = 1.0 - beta_t
    sab = jnp.stack([jnp.sqrt(alpha), jnp.sqrt(beta_t)]).reshape(1, 2)
    sab = sab.astype(jnp.float32)

    pospad = jnp.pad(pos, ((0, 0), (0, 1)))
    noisepad = jnp.pad(noise, ((0, 0), (0, 1)))
    z128 = jnp.zeros((N, D), jnp.float32)
    z4flat = jnp.zeros((N * 4,), jnp.float32)

    P, A, B = _tc_prep(pospad, noisepad, x, We1[0, :D, :], We1[0, D:2 * D, :],
                       sab)
    h = x
    for l in range(L):
        last = l == L - 1
        pre, relflat = _sc_gather(A, B, P.reshape(N * 4), src, dst)
        rel = relflat.reshape(E, 4)
        wc = We1[l, 2 * D, :].reshape(1, D)
        outs = _tc_edge(pre, rel, wc, be1[l].reshape(1, D), We2[l],
                        be2[l].reshape(1, D), Wc1[l], bc1[l].reshape(1, D),
                        Wc2[l].reshape(1, D), bc2[l].reshape(1, 1),
                        write_m=not last)
        if last:
            (wr,) = outs
            apflat = _sc_scatter_pos(wr.reshape(E * 4), dst, z4flat, N)
            ap = _tc_reduce_pos(apflat.reshape(_NW, N * 4)).reshape(N, 4)
            P = _tc_pos_update(P, ap)
        else:
            m, wr = outs
            am2 = _sc_scatter_m(m, dst, z128)
            apflat = _sc_scatter_pos(wr.reshape(E * 4), dst, z4flat, N)
            ap = _tc_reduce_pos(apflat.reshape(_NW, N * 4)).reshape(N, 4)
            h, P, A, B = _tc_node(h, P, am2, ap, Wn1[l, :D, :],
                                  Wn1[l, D:, :], bn1[l].reshape(1, D),
                                  Wn2[l], bn2[l].reshape(1, D),
                                  We1[l + 1, :D, :], We1[l + 1, D:2 * D, :])
    return P[:, :3]


# speedup vs baseline: 3.6896x; 1.1766x over previous
import dataclasses
import functools

import jax
import jax.numpy as jnp
from jax import lax
from jax.experimental import pallas as pl
from jax.experimental.pallas import tpu as pltpu
from jax.experimental.pallas import tpu_sc as plsc

_NC = 2
_NS = 16
_NW = _NC * _NS
_CH = 128
_PREC = jax.lax.Precision.HIGHEST


def _silu(v):
    return v * (1.0 / (1.0 + jnp.exp(-v)))


def _mesh():
    return plsc.VectorSubcoreMesh(core_axis_name="c", subcore_axis_name="s")


def _sc_params():
    cp = pltpu.CompilerParams()
    if "needs_layout_passes" in pltpu.CompilerParams.__dataclass_fields__:
        cp = dataclasses.replace(cp, needs_layout_passes=False)
    return cp


def _dot(a, b):
    return jnp.dot(a.astype(jnp.bfloat16), b.astype(jnp.bfloat16),
                   preferred_element_type=jnp.float32)


def _b(v):
    return v.astype(jnp.bfloat16).astype(jnp.float32)


def _sc_gather(A, B, p4flat, srcT, dstT, E):
    N, D = A.shape
    nch = E // _CH
    npass = srcT.shape[1]
    assert npass % 2 == 0

    @functools.partial(
        pl.kernel,
        out_type=(jax.ShapeDtypeStruct((E, D), jnp.float32),
                  jax.ShapeDtypeStruct((E * 4,), jnp.float32)),
        mesh=_mesh(),
        scratch_types=[pltpu.VMEM((npass, _CH), jnp.int32),
                       pltpu.VMEM((npass, _CH), jnp.int32),
                       pltpu.VMEM((_CH, D), jnp.float32),
                       pltpu.VMEM((_CH, D), jnp.float32),
                       pltpu.VMEM((_CH, D), jnp.float32),
                       pltpu.VMEM((_CH, D), jnp.float32),
                       pltpu.VMEM((_CH * 4,), jnp.float32),
                       pltpu.VMEM((_CH * 4,), jnp.float32),
                       pltpu.VMEM((N * 4,), jnp.float32),
                       pltpu.SemaphoreType.DMA,
                       pltpu.SemaphoreType.DMA,
                       pltpu.SemaphoreType.DMA,
                       pltpu.SemaphoreType.DMA],
        compiler_params=_sc_params(),
    )
    def k(a_hbm, b_hbm, p4_hbm, srcT_hbm, dstT_hbm, pre_hbm, rel_hbm,
          iva, ivb, ba0, bb0, ba1, bb1, relv0, relv1, p4v,
          sg0, sg1, sw0, sw1):
        w = lax.axis_index("c") * _NS + lax.axis_index("s")
        pltpu.sync_copy(p4_hbm, p4v)
        pltpu.sync_copy(srcT_hbm.at[w], iva)
        pltpu.sync_copy(dstT_hbm.at[w], ivb)
        zero16 = jnp.zeros((16,), jnp.float32)

        @pl.loop(0, _CH * 4, step=16)
        def _(i):
            relv0[pl.ds(i, 16)] = zero16
            relv1[pl.ds(i, 16)] = zero16

        iota = lax.iota(jnp.int32, 16)
        bufs = ((ba0, bb0, relv0, sg0, sw0), (ba1, bb1, relv1, sg1, sw1))

        def start_gather(kk, buf):
            ba, bb, _relv, sg, _sw = buf
            pltpu.make_async_copy(a_hbm.at[ivb.at[kk]], ba, sg).start()
            pltpu.make_async_copy(b_hbm.at[iva.at[kk]], bb, sg).start()

        def drain_writes(g, buf):
            ba, _bb, relv, _sg, sw = buf
            off = g * _CH
            pltpu.make_async_copy(ba, pre_hbm.at[pl.ds(off, _CH)], sw).wait()
            pltpu.make_async_copy(relv, rel_hbm.at[pl.ds(off * 4, _CH * 4)],
                                  sw).wait()

        start_gather(0, bufs[0])

        @pl.loop(0, npass // 2)
        def _(ii):
            for b in range(2):
                kk = ii * 2 + b
                g = kk * _NW + w
                cur = bufs[b]
                nxt = bufs[1 - b]
                gn = g + _NW

                @pl.when(jnp.logical_and(kk + 1 < npass, gn < nch))
                def _():
                    @pl.when(kk >= 1)
                    def _():
                        drain_writes(gn - 2 * _NW, nxt)
                    start_gather(kk + 1, nxt)

                @pl.when(g < nch)
                def _():
                    ba, bb, relv, sg, sw = cur

                    @pl.loop(0, _CH // 16)
                    def _(j):
                        d16 = ivb[kk, pl.ds(j * 16, 16)] * 4
                        s16 = iva[kk, pl.ds(j * 16, 16)] * 4
                        out_ix = j * 64 + iota * 4
                        for c in range(3):
                            pd = plsc.load_gather(p4v, [d16 + c])
                            ps = plsc.load_gather(p4v, [s16 + c])
                            plsc.store_scatter(relv, [out_ix + c], pd - ps)

                    pltpu.make_async_copy(a_hbm.at[ivb.at[kk]], ba, sg).wait()
                    pltpu.make_async_copy(b_hbm.at[iva.at[kk]], bb, sg).wait()

                    @pl.loop(0, _CH)
                    def _(r):
                        for cb in range(D // 16):
                            s_ = pl.ds(cb * 16, 16)
                            ba[r, s_] = ba[r, s_] + bb[r, s_]

                    off = g * _CH
                    pltpu.make_async_copy(ba, pre_hbm.at[pl.ds(off, _CH)],
                                          sw).start()
                    pltpu.make_async_copy(relv,
                                          rel_hbm.at[pl.ds(off * 4, _CH * 4)],
                                          sw).start()

        for kk in (npass - 2, npass - 1):
            g_l = kk * _NW + w

            @pl.when(g_l < nch)
            def _():
                drain_writes(g_l, bufs[kk % 2])

    return k(A, B, p4flat, srcT, dstT)


def _sc_scatter_m(m, dst, z128):
    E, D = m.shape
    N = z128.shape[0]
    nch = E // _CH
    npass = (nch + _NW - 1) // _NW
    rpt = (N // _NS) // 8 * 8
    tail = N - rpt * _NS

    @functools.partial(
        pl.kernel,
        out_type=jax.ShapeDtypeStruct((_NC * N, D), jnp.float32),
        mesh=_mesh(),
        scratch_types=[pltpu.VMEM((_CH,), jnp.int32),
                       pltpu.VMEM((_CH, D), jnp.float32),
                       pltpu.VMEM_SHARED((N, D), jnp.float32),
                       pltpu.SemaphoreType.DMA],
        compiler_params=_sc_params(),
    )
    def k(m_hbm, dst_hbm, z128_hbm, am_hbm, iv, mv, shm, sem):
        c = lax.axis_index("c")
        s = lax.axis_index("s")
        w_ = c * _NS + s
        r0 = s * rpt
        pltpu.sync_copy(z128_hbm.at[pl.ds(r0, rpt)], shm.at[pl.ds(r0, rpt)])

        @pl.when(s == 0)
        def _():
            pltpu.sync_copy(z128_hbm.at[pl.ds(rpt * _NS, tail)],
                            shm.at[pl.ds(rpt * _NS, tail)])

        plsc.subcore_barrier()

        @pl.loop(0, npass)
        def _(i):
            g = i * _NW + w_

            @pl.when(g < nch)
            def _():
                off = g * _CH
                pltpu.sync_copy(dst_hbm.at[pl.ds(off, _CH)], iv)
                pltpu.sync_copy(m_hbm.at[pl.ds(off, _CH)], mv)
                pltpu.sync_copy(mv, shm.at[iv], add=True)

        plsc.subcore_barrier()
        pltpu.sync_copy(shm.at[pl.ds(r0, rpt)],
                        am_hbm.at[pl.ds(c * N + r0, rpt)])

        @pl.when(s == 0)
        def _():
            pltpu.sync_copy(shm.at[pl.ds(rpt * _NS, tail)],
                            am_hbm.at[pl.ds(c * N + rpt * _NS, tail)])

    return k(m, dst, z128)


def _sc_scatter_pos(wrel4flat, dst, z4flat, N):
    E4 = wrel4flat.shape[0]
    E = E4 // 4
    nch = E // _CH
    npass = (nch + _NW - 1) // _NW

    @functools.partial(
        pl.kernel,
        out_type=jax.ShapeDtypeStruct((_NW * N * 4,), jnp.float32),
        mesh=_mesh(),
        scratch_types=[pltpu.VMEM((_CH,), jnp.int32),
                       pltpu.VMEM((_CH * 4,), jnp.float32),
                       pltpu.VMEM((N * 4,), jnp.float32),
                       pltpu.SemaphoreType.DMA],
        compiler_params=_sc_params(),
    )
    def k(w_hbm, dst_hbm, z4_hbm, ap_hbm, iv, wv, pacc, sem):
        c = lax.axis_index("c")
        s = lax.axis_index("s")
        w_ = c * _NS + s
        pltpu.sync_copy(z4_hbm, pacc)
        iota = lax.iota(jnp.int32, 16)

        @pl.loop(0, npass)
        def _(i):
            g = i * _NW + w_

            @pl.when(g < nch)
            def _():
                off = g * _CH
                pltpu.sync_copy(dst_hbm.at[pl.ds(off, _CH)], iv)
                pltpu.sync_copy(w_hbm.at[pl.ds(off * 4, _CH * 4)], wv)

                @pl.loop(0, _CH // 16)
                def _(j):
                    d16 = iv[pl.ds(j * 16, 16)] * 4
                    in_ix = j * 64 + iota * 4
                    for cc in range(4):
                        val = plsc.load_gather(wv, [in_ix + cc])
                        plsc.addupdate_scatter(pacc, [d16 + cc], val)

        pltpu.sync_copy(pacc, ap_hbm.at[pl.ds(w_ * (N * 4), N * 4)])

    return k(wrel4flat, dst, z4flat)


def _full(shape):
    return pl.BlockSpec(shape, lambda i: (0,) * len(shape))


def _tc_prep(pospad, noisepad, x, Wa, Wb, sab):
    N, D = x.shape
    BN = 2000

    def body(pp, np_, x_, wa, wb, sab_, p_o, a_o, b_o):
        p_o[...] = sab_[0, 0] * pp[...] + sab_[0, 1] * np_[...]
        a_o[...] = _dot(x_[...], wa[...])
        b_o[...] = _dot(x_[...], wb[...])

    row = pl.BlockSpec((BN, 4), lambda i: (i, 0))
    big = pl.BlockSpec((BN, D), lambda i: (i, 0))
    return pl.pallas_call(
        body,
        grid=(N // BN,),
        in_specs=[row, row, big, _full((D, D)), _full((D, D)), _full((1, 2))],
        out_specs=[row, big, big],
        out_shape=(jax.ShapeDtypeStruct((N, 4), jnp.float32),
                   jax.ShapeDtypeStruct((N, D), jnp.float32),
                   jax.ShapeDtypeStruct((N, D), jnp.float32)),
    )(pospad, noisepad, x, Wa, Wb, sab)


def _tc_edge(pre, rel, wc, b1, We2l, b2, Wc1l, bc1l, wc2r, bc2l, write_m):
    E, D = pre.shape
    BE = 2000

    def body(pre_ref, rel_ref, wc_ref, b1_ref, w2_ref, b2_ref, wc1_ref,
             bc1_ref, wc2_ref, bc2_ref, *outs):
        relv = rel_ref[...]
        d2 = jnp.sum(relv * relv, axis=1, keepdims=True)
        m1 = _silu(pre_ref[...] + _b(d2) * _b(wc_ref[...]) + b1_ref[...])
        m = _silu(_dot(m1, w2_ref[...]) + b2_ref[...])
        c1 = _silu(_dot(m, wc1_ref[...]) + bc1_ref[...])
        coef = (jnp.sum(_b(c1) * _b(wc2_ref[...]), axis=1, keepdims=True)
                + bc2_ref[0, 0])
        lane = lax.broadcasted_iota(jnp.int32, (BE, 4), 1)
        wrel = relv * coef + (lane == 3).astype(jnp.float32)
        if write_m:
            outs[0][...] = m
            outs[1][...] = wrel
        else:
            outs[0][...] = wrel

    big = pl.BlockSpec((BE, D), lambda i: (i, 0))
    row = pl.BlockSpec((BE, 4), lambda i: (i, 0))
    wspec = _full((D, D))
    vspec = _full((1, D))
    out_specs = [row]
    out_shape = [jax.ShapeDtypeStruct((E, 4), jnp.float32)]
    if write_m:
        out_specs = [big] + out_specs
        out_shape = [jax.ShapeDtypeStruct((E, D), jnp.float32)] + out_shape
    return pl.pallas_call(
        body,
        grid=(E // BE,),
        in_specs=[big, row, vspec, vspec, wspec, vspec, wspec, vspec, vspec,
                  _full((1, 1))],
        out_specs=out_specs,
        out_shape=tuple(out_shape),
    )(pre, rel, wc, b1, We2l, b2, Wc1l, bc1l, wc2r, bc2l)


def _tc_reduce_pos(ap2d):
    NW, F = ap2d.shape

    def body(ap_ref, o_ref):
        o_ref[...] = jnp.sum(ap_ref[...], axis=0)

    return pl.pallas_call(
        body,
        grid=(1,),
        in_specs=[pl.BlockSpec((NW, F), lambda i: (0, 0))],
        out_specs=pl.BlockSpec((F,), lambda i: (0,)),
        out_shape=jax.ShapeDtypeStruct((F,), jnp.float32),
    )(ap2d)


def _tc_node(h, P, am2, ap, Wn1a, Wn1b, b1, Wn2l, b2, Wa, Wb):
    N, D = h.shape
    BN = 2000
    nb = N // BN
    with_ab = Wa is not None

    def body(h_ref, p_ref, am0, am1, ap_ref, w1a, w1b, b1_ref, w2, b2_ref,
             *rest):
        if with_ab:
            wa, wb, h_o, p_o, a_o, b_o = rest
        else:
            h_o, p_o, a_o, b_o = rest[:2] + (None, None)
        aggm = am0[...] + am1[...]
        aggp = ap_ref[...]
        deg = jnp.maximum(aggp[:, 3:4], 1.0)
        p_o[...] = p_ref[...] + aggp / deg
        u = _silu(_dot(h_ref[...], w1a[...]) + _dot(aggm, w1b[...])
                  + b1_ref[...])
        hn = h_ref[...] + _dot(u, w2[...]) + b2_ref[...]
        h_o[...] = hn
        if with_ab:
            a_o[...] = _dot(hn, wa[...])
            b_o[...] = _dot(hn, wb[...])

    big = pl.BlockSpec((BN, D), lambda i: (i, 0))
    row = pl.BlockSpec((BN, 4), lambda i: (i, 0))
    big1 = pl.BlockSpec((BN, D), lambda i: (i + nb, 0))
    wspec = _full((D, D))
    vspec = _full((1, D))
    in_specs = [big, row, big, big1, row, wspec, wspec, vspec,
                wspec, vspec]
    args = [h, P, am2, am2, ap, Wn1a, Wn1b, b1, Wn2l, b2]
    out_specs = [big, row]
    out_shape = [jax.ShapeDtypeStruct((N, D), jnp.float32),
                 jax.ShapeDtypeStruct((N, 4), jnp.float32)]
    if with_ab:
        in_specs += [wspec, wspec]
        args += [Wa, Wb]
        out_specs += [big, big]
        out_shape += [jax.ShapeDtypeStruct((N, D), jnp.float32),
                      jax.ShapeDtypeStruct((N, D), jnp.float32)]
    return pl.pallas_call(
        body,
        grid=(nb,),
        in_specs=in_specs,
        out_specs=out_specs,
        out_shape=tuple(out_shape),
    )(*args)


def _tc_pos_update(P, ap):
    N = P.shape[0]
    BN = 2000
    nb = N // BN

    def body(p_ref, ap_ref, p_o):
        aggp = ap_ref[...]
        deg = jnp.maximum(aggp[:, 3:4], 1.0)
        p_o[...] = p_ref[...] + aggp / deg

    row = pl.BlockSpec((BN, 4), lambda i: (i, 0))
    return pl.pallas_call(
        body,
        grid=(nb,),
        in_specs=[row, row],
        out_specs=row,
        out_shape=jax.ShapeDtypeStruct((N, 4), jnp.float32),
    )(P, ap)


def kernel(x, pos, edge_index, t, noise, We1, be1, We2, be2, Wc1, bc1, Wc2,
           bc2, Wn1, bn1, Wn2, bn2):
    N, D = x.shape
    E = edge_index.shape[1]
    L = We1.shape[0]
    src = edge_index[0]
    dst = edge_index[1]

    beta_t = 0.0001 + (0.02 - 0.0001) * (jnp.float32(t) / 999.0)
    alpha = 1.0 - beta_t
    sab = jnp.stack([jnp.sqrt(alpha), jnp.sqrt(beta_t)]).reshape(1, 2)
    sab = sab.astype(jnp.float32)

    pospad = jnp.pad(pos, ((0, 0), (0, 1)))
    noisepad = jnp.pad(noise, ((0, 0), (0, 1)))
    z128 = jnp.zeros((N, D), jnp.float32)
    z4flat = jnp.zeros((N * 4,), jnp.float32)

    nch = E // _CH
    npass = -(-nch // _NW)
    pad = npass * _NW * _CH - E
    srcT = jnp.pad(src, (0, pad)).reshape(npass, _NW, _CH).transpose(1, 0, 2)
    dstT = jnp.pad(dst, (0, pad)).reshape(npass, _NW, _CH).transpose(1, 0, 2)

    P, A, B = _tc_prep(pospad, noisepad, x, We1[0, :D, :], We1[0, D:2 * D, :],
                       sab)
    h = x
    for l in range(L):
        last = l == L - 1
        pre, relflat = _sc_gather(A, B, P.reshape(N * 4), srcT, dstT, E)
        rel = relflat.reshape(E, 4)
        wc = We1[l, 2 * D, :].reshape(1, D)
        outs = _tc_edge(pre, rel, wc, be1[l].reshape(1, D), We2[l],
                        be2[l].reshape(1, D), Wc1[l], bc1[l].reshape(1, D),
                        Wc2[l].reshape(1, D), bc2[l].reshape(1, 1),
                        write_m=not last)
        if last:
            (wr,) = outs
            apflat = _sc_scatter_pos(wr.reshape(E * 4), dst, z4flat, N)
            ap = _tc_reduce_pos(apflat.reshape(_NW, N * 4)).reshape(N, 4)
            P = _tc_pos_update(P, ap)
        else:
            m, wr = outs
            am2 = _sc_scatter_m(m, dst, z128)
            apflat = _sc_scatter_pos(wr.reshape(E * 4), dst, z4flat, N)
            ap = _tc_reduce_pos(apflat.reshape(_NW, N * 4)).reshape(N, 4)
            h, P, A, B = _tc_node(h, P, am2, ap, Wn1[l, :D, :],
                                  Wn1[l, D:, :], bn1[l].reshape(1, D),
                                  Wn2[l], bn2[l].reshape(1, D),
                                  We1[l + 1, :D, :], We1[l + 1, D:2 * D, :])
    return P[:, :3]


# speedup vs baseline: 4.1026x; 1.1119x over previous
import dataclasses
import functools

import jax
import jax.numpy as jnp
from jax import lax
from jax.experimental import pallas as pl
from jax.experimental.pallas import tpu as pltpu
from jax.experimental.pallas import tpu_sc as plsc

_NC = 2
_NS = 16
_NW = _NC * _NS
_CH = 128
_PREC = jax.lax.Precision.HIGHEST


def _silu(v):
    return v * (1.0 / (1.0 + jnp.exp(-v)))


def _mesh():
    return plsc.VectorSubcoreMesh(core_axis_name="c", subcore_axis_name="s")


def _sc_params():
    cp = pltpu.CompilerParams()
    if "needs_layout_passes" in pltpu.CompilerParams.__dataclass_fields__:
        cp = dataclasses.replace(cp, needs_layout_passes=False)
    return cp


def _dot(a, b):
    return jnp.dot(a.astype(jnp.bfloat16), b.astype(jnp.bfloat16),
                   preferred_element_type=jnp.float32)


def _b(v):
    return v.astype(jnp.bfloat16).astype(jnp.float32)


def _sc_gather(A, B, p4flat, srcT, dstT, E):
    N, D = A.shape
    nch = E // _CH
    npass = srcT.shape[1]
    assert npass % 2 == 0

    @functools.partial(
        pl.kernel,
        out_type=(jax.ShapeDtypeStruct((E, D), jnp.float32),
                  jax.ShapeDtypeStruct((E * 4,), jnp.float32)),
        mesh=_mesh(),
        scratch_types=[pltpu.VMEM((npass, _CH), jnp.int32),
                       pltpu.VMEM((npass, _CH), jnp.int32),
                       pltpu.VMEM((_CH, D), jnp.float32),
                       pltpu.VMEM((_CH, D), jnp.float32),
                       pltpu.VMEM((_CH, D), jnp.float32),
                       pltpu.VMEM((_CH, D), jnp.float32),
                       pltpu.VMEM((_CH * 4,), jnp.float32),
                       pltpu.VMEM((_CH * 4,), jnp.float32),
                       pltpu.VMEM((N * 4,), jnp.float32),
                       pltpu.SemaphoreType.DMA,
                       pltpu.SemaphoreType.DMA,
                       pltpu.SemaphoreType.DMA,
                       pltpu.SemaphoreType.DMA],
        compiler_params=_sc_params(),
    )
    def k(a_hbm, b_hbm, p4_hbm, srcT_hbm, dstT_hbm, pre_hbm, rel_hbm,
          iva, ivb, ba0, bb0, ba1, bb1, relv0, relv1, p4v,
          sg0, sg1, sw0, sw1):
        w = lax.axis_index("c") * _NS + lax.axis_index("s")
        pltpu.sync_copy(p4_hbm, p4v)
        pltpu.sync_copy(srcT_hbm.at[w], iva)
        pltpu.sync_copy(dstT_hbm.at[w], ivb)
        zero16 = jnp.zeros((16,), jnp.float32)

        @pl.loop(0, _CH * 4, step=16)
        def _(i):
            relv0[pl.ds(i, 16)] = zero16
            relv1[pl.ds(i, 16)] = zero16

        iota = lax.iota(jnp.int32, 16)
        bufs = ((ba0, bb0, relv0, sg0, sw0), (ba1, bb1, relv1, sg1, sw1))

        def start_gather(kk, buf):
            ba, bb, _relv, sg, _sw = buf
            pltpu.make_async_copy(a_hbm.at[ivb.at[kk]], ba, sg).start()
            pltpu.make_async_copy(b_hbm.at[iva.at[kk]], bb, sg).start()

        def drain_writes(g, buf):
            ba, _bb, relv, _sg, sw = buf
            off = g * _CH
            pltpu.make_async_copy(ba, pre_hbm.at[pl.ds(off, _CH)], sw).wait()
            pltpu.make_async_copy(relv, rel_hbm.at[pl.ds(off * 4, _CH * 4)],
                                  sw).wait()

        start_gather(0, bufs[0])

        @pl.loop(0, npass // 2)
        def _(ii):
            for b in range(2):
                kk = ii * 2 + b
                g = kk * _NW + w
                cur = bufs[b]
                nxt = bufs[1 - b]
                gn = g + _NW

                @pl.when(jnp.logical_and(kk + 1 < npass, gn < nch))
                def _():
                    @pl.when(kk >= 1)
                    def _():
                        drain_writes(gn - 2 * _NW, nxt)
                    start_gather(kk + 1, nxt)

                @pl.when(g < nch)
                def _():
                    ba, bb, relv, sg, sw = cur

                    @pl.loop(0, _CH // 16)
                    def _(j):
                        d16 = ivb[kk, pl.ds(j * 16, 16)] * 4
                        s16 = iva[kk, pl.ds(j * 16, 16)] * 4
                        out_ix = j * 64 + iota * 4
                        for c in range(3):
                            pd = plsc.load_gather(p4v, [d16 + c])
                            ps = plsc.load_gather(p4v, [s16 + c])
                            plsc.store_scatter(relv, [out_ix + c], pd - ps)

                    pltpu.make_async_copy(a_hbm.at[ivb.at[kk]], ba, sg).wait()
                    pltpu.make_async_copy(b_hbm.at[iva.at[kk]], bb, sg).wait()

                    @pl.loop(0, _CH)
                    def _(r):
                        for cb in range(D // 16):
                            s_ = pl.ds(cb * 16, 16)
                            ba[r, s_] = ba[r, s_] + bb[r, s_]

                    off = g * _CH
                    pltpu.make_async_copy(ba, pre_hbm.at[pl.ds(off, _CH)],
                                          sw).start()
                    pltpu.make_async_copy(relv,
                                          rel_hbm.at[pl.ds(off * 4, _CH * 4)],
                                          sw).start()

        for kk in (npass - 2, npass - 1):
            g_l = kk * _NW + w

            @pl.when(g_l < nch)
            def _():
                drain_writes(g_l, bufs[kk % 2])

    return k(A, B, p4flat, srcT, dstT)


def _sc_scatter_m(m, dstT, z128, E):
    D = m.shape[1]
    N = z128.shape[0]
    nch = E // _CH
    npass = dstT.shape[1]
    assert npass % 2 == 0
    rpt = (N // _NS) // 8 * 8
    tail = N - rpt * _NS

    @functools.partial(
        pl.kernel,
        out_type=jax.ShapeDtypeStruct((_NC * N, D), jnp.float32),
        mesh=_mesh(),
        scratch_types=[pltpu.VMEM((npass, _CH), jnp.int32),
                       pltpu.VMEM((_CH, D), jnp.float32),
                       pltpu.VMEM((_CH, D), jnp.float32),
                       pltpu.VMEM_SHARED((N, D), jnp.float32),
                       pltpu.SemaphoreType.DMA,
                       pltpu.SemaphoreType.DMA],
        compiler_params=_sc_params(),
    )
    def k(m_hbm, dstT_hbm, z128_hbm, am_hbm, ivb, mv0, mv1, shm, sm0, sm1):
        c = lax.axis_index("c")
        s = lax.axis_index("s")
        w_ = c * _NS + s
        r0 = s * rpt
        pltpu.sync_copy(z128_hbm.at[pl.ds(r0, rpt)], shm.at[pl.ds(r0, rpt)])

        @pl.when(s == 0)
        def _():
            pltpu.sync_copy(z128_hbm.at[pl.ds(rpt * _NS, tail)],
                            shm.at[pl.ds(rpt * _NS, tail)])

        pltpu.sync_copy(dstT_hbm.at[w_], ivb)
        plsc.subcore_barrier()
        bufs = ((mv0, sm0), (mv1, sm1))

        def start_m(g, buf):
            mv, sm = buf
            pltpu.make_async_copy(m_hbm.at[pl.ds(g * _CH, _CH)], mv,
                                  sm).start()

        start_m(w_, bufs[0])

        @pl.loop(0, npass // 2)
        def _(ii):
            for b in range(2):
                kk = ii * 2 + b
                g = kk * _NW + w_
                mv, sm = bufs[b]
                gn = g + _NW

                @pl.when(jnp.logical_and(kk + 1 < npass, gn < nch))
                def _():
                    start_m(gn, bufs[1 - b])

                @pl.when(g < nch)
                def _():
                    pltpu.make_async_copy(m_hbm.at[pl.ds(g * _CH, _CH)], mv,
                                          sm).wait()
                    pltpu.sync_copy(mv, shm.at[ivb.at[kk]], add=True)

        plsc.subcore_barrier()
        pltpu.sync_copy(shm.at[pl.ds(r0, rpt)],
                        am_hbm.at[pl.ds(c * N + r0, rpt)])

        @pl.when(s == 0)
        def _():
            pltpu.sync_copy(shm.at[pl.ds(rpt * _NS, tail)],
                            am_hbm.at[pl.ds(c * N + rpt * _NS, tail)])

    return k(m, dstT, z128)


def _sc_scatter_pos(wrel4flat, dstT, z4flat, N, E):
    nch = E // _CH
    npass = dstT.shape[1]
    assert npass % 2 == 0

    @functools.partial(
        pl.kernel,
        out_type=jax.ShapeDtypeStruct((_NW * N * 4,), jnp.float32),
        mesh=_mesh(),
        scratch_types=[pltpu.VMEM((npass, _CH), jnp.int32),
                       pltpu.VMEM((_CH * 4,), jnp.float32),
                       pltpu.VMEM((_CH * 4,), jnp.float32),
                       pltpu.VMEM((N * 4,), jnp.float32),
                       pltpu.SemaphoreType.DMA,
                       pltpu.SemaphoreType.DMA],
        compiler_params=_sc_params(),
    )
    def k(w_hbm, dstT_hbm, z4_hbm, ap_hbm, ivb, wv0, wv1, pacc, sw0, sw1):
        c = lax.axis_index("c")
        s = lax.axis_index("s")
        w_ = c * _NS + s
        pltpu.sync_copy(z4_hbm, pacc)
        pltpu.sync_copy(dstT_hbm.at[w_], ivb)
        iota = lax.iota(jnp.int32, 16)
        bufs = ((wv0, sw0), (wv1, sw1))

        def start_w(g, buf):
            wv, sw = buf
            pltpu.make_async_copy(w_hbm.at[pl.ds(g * _CH * 4, _CH * 4)], wv,
                                  sw).start()

        start_w(w_, bufs[0])

        @pl.loop(0, npass // 2)
        def _(ii):
            for b in range(2):
                kk = ii * 2 + b
                g = kk * _NW + w_
                wv, sw = bufs[b]
                gn = g + _NW

                @pl.when(jnp.logical_and(kk + 1 < npass, gn < nch))
                def _():
                    start_w(gn, bufs[1 - b])

                @pl.when(g < nch)
                def _():
                    pltpu.make_async_copy(
                        w_hbm.at[pl.ds(g * _CH * 4, _CH * 4)], wv, sw).wait()

                    @pl.loop(0, _CH // 16)
                    def _(j):
                        d16 = ivb[kk, pl.ds(j * 16, 16)] * 4
                        in_ix = j * 64 + iota * 4
                        for cc in range(4):
                            val = plsc.load_gather(wv, [in_ix + cc])
                            plsc.addupdate_scatter(pacc, [d16 + cc], val)

        pltpu.sync_copy(pacc, ap_hbm.at[pl.ds(w_ * (N * 4), N * 4)])

    return k(wrel4flat, dstT, z4flat)


def _full(shape):
    return pl.BlockSpec(shape, lambda i: (0,) * len(shape))


def _tc_prep(pospad, noisepad, x, Wa, Wb, sab):
    N, D = x.shape
    BN = 2000

    def body(pp, np_, x_, wa, wb, sab_, p_o, a_o, b_o):
        p_o[...] = sab_[0, 0] * pp[...] + sab_[0, 1] * np_[...]
        a_o[...] = _dot(x_[...], wa[...])
        b_o[...] = _dot(x_[...], wb[...])

    row = pl.BlockSpec((BN, 4), lambda i: (i, 0))
    big = pl.BlockSpec((BN, D), lambda i: (i, 0))
    return pl.pallas_call(
        body,
        grid=(N // BN,),
        in_specs=[row, row, big, _full((D, D)), _full((D, D)), _full((1, 2))],
        out_specs=[row, big, big],
        out_shape=(jax.ShapeDtypeStruct((N, 4), jnp.float32),
                   jax.ShapeDtypeStruct((N, D), jnp.float32),
                   jax.ShapeDtypeStruct((N, D), jnp.float32)),
    )(pospad, noisepad, x, Wa, Wb, sab)


def _tc_edge(pre, rel, wc, b1, We2l, b2, Wc1l, bc1l, wc2r, bc2l, write_m):
    E, D = pre.shape
    BE = 2000

    def body(pre_ref, rel_ref, wc_ref, b1_ref, w2_ref, b2_ref, wc1_ref,
             bc1_ref, wc2_ref, bc2_ref, *outs):
        relv = rel_ref[...]
        d2 = jnp.sum(relv * relv, axis=1, keepdims=True)
        m1 = _silu(pre_ref[...] + _b(d2) * _b(wc_ref[...]) + b1_ref[...])
        m = _silu(_dot(m1, w2_ref[...]) + b2_ref[...])
        c1 = _silu(_dot(m, wc1_ref[...]) + bc1_ref[...])
        coef = (jnp.sum(_b(c1) * _b(wc2_ref[...]), axis=1, keepdims=True)
                + bc2_ref[0, 0])
        lane = lax.broadcasted_iota(jnp.int32, (BE, 4), 1)
        wrel = relv * coef + (lane == 3).astype(jnp.float32)
        if write_m:
            outs[0][...] = m
            outs[1][...] = wrel
        else:
            outs[0][...] = wrel

    big = pl.BlockSpec((BE, D), lambda i: (i, 0))
    row = pl.BlockSpec((BE, 4), lambda i: (i, 0))
    wspec = _full((D, D))
    vspec = _full((1, D))
    out_specs = [row]
    out_shape = [jax.ShapeDtypeStruct((E, 4), jnp.float32)]
    if write_m:
        out_specs = [big] + out_specs
        out_shape = [jax.ShapeDtypeStruct((E, D), jnp.float32)] + out_shape
    return pl.pallas_call(
        body,
        grid=(E // BE,),
        in_specs=[big, row, vspec, vspec, wspec, vspec, wspec, vspec, vspec,
                  _full((1, 1))],
        out_specs=out_specs,
        out_shape=tuple(out_shape),
    )(pre, rel, wc, b1, We2l, b2, Wc1l, bc1l, wc2r, bc2l)


def _tc_reduce_pos(ap2d):
    NW, F = ap2d.shape

    def body(ap_ref, o_ref):
        o_ref[...] = jnp.sum(ap_ref[...], axis=0)

    return pl.pallas_call(
        body,
        grid=(1,),
        in_specs=[pl.BlockSpec((NW, F), lambda i: (0, 0))],
        out_specs=pl.BlockSpec((F,), lambda i: (0,)),
        out_shape=jax.ShapeDtypeStruct((F,), jnp.float32),
    )(ap2d)


def _tc_node(h, P, am2, ap, Wn1a, Wn1b, b1, Wn2l, b2, Wa, Wb):
    N, D = h.shape
    BN = 2000
    nb = N // BN
    with_ab = Wa is not None

    def body(h_ref, p_ref, am0, am1, ap_ref, w1a, w1b, b1_ref, w2, b2_ref,
             *rest):
        if with_ab:
            wa, wb, h_o, p_o, a_o, b_o = rest
        else:
            h_o, p_o, a_o, b_o = rest[:2] + (None, None)
        aggm = am0[...] + am1[...]
        aggp = ap_ref[...]
        deg = jnp.maximum(aggp[:, 3:4], 1.0)
        p_o[...] = p_ref[...] + aggp / deg
        u = _silu(_dot(h_ref[...], w1a[...]) + _dot(aggm, w1b[...])
                  + b1_ref[...])
        hn = h_ref[...] + _dot(u, w2[...]) + b2_ref[...]
        h_o[...] = hn
        if with_ab:
            a_o[...] = _dot(hn, wa[...])
            b_o[...] = _dot(hn, wb[...])

    big = pl.BlockSpec((BN, D), lambda i: (i, 0))
    row = pl.BlockSpec((BN, 4), lambda i: (i, 0))
    big1 = pl.BlockSpec((BN, D), lambda i: (i + nb, 0))
    wspec = _full((D, D))
    vspec = _full((1, D))
    in_specs = [big, row, big, big1, row, wspec, wspec, vspec,
                wspec, vspec]
    args = [h, P, am2, am2, ap, Wn1a, Wn1b, b1, Wn2l, b2]
    out_specs = [big, row]
    out_shape = [jax.ShapeDtypeStruct((N, D), jnp.float32),
                 jax.ShapeDtypeStruct((N, 4), jnp.float32)]
    if with_ab:
        in_specs += [wspec, wspec]
        args += [Wa, Wb]
        out_specs += [big, big]
        out_shape += [jax.ShapeDtypeStruct((N, D), jnp.float32),
                      jax.ShapeDtypeStruct((N, D), jnp.float32)]
    return pl.pallas_call(
        body,
        grid=(nb,),
        in_specs=in_specs,
        out_specs=out_specs,
        out_shape=tuple(out_shape),
    )(*args)


def _tc_pos_update(P, ap):
    N = P.shape[0]
    BN = 2000
    nb = N // BN

    def body(p_ref, ap_ref, p_o):
        aggp = ap_ref[...]
        deg = jnp.maximum(aggp[:, 3:4], 1.0)
        p_o[...] = p_ref[...] + aggp / deg

    row = pl.BlockSpec((BN, 4), lambda i: (i, 0))
    return pl.pallas_call(
        body,
        grid=(nb,),
        in_specs=[row, row],
        out_specs=row,
        out_shape=jax.ShapeDtypeStruct((N, 4), jnp.float32),
    )(P, ap)


def kernel(x, pos, edge_index, t, noise, We1, be1, We2, be2, Wc1, bc1, Wc2,
           bc2, Wn1, bn1, Wn2, bn2):
    N, D = x.shape
    E = edge_index.shape[1]
    L = We1.shape[0]
    src = edge_index[0]
    dst = edge_index[1]

    beta_t = 0.0001 + (0.02 - 0.0001) * (jnp.float32(t) / 999.0)
    alpha = 1.0 - beta_t
    sab = jnp.stack([jnp.sqrt(alpha), jnp.sqrt(beta_t)]).reshape(1, 2)
    sab = sab.astype(jnp.float32)

    pospad = jnp.pad(pos, ((0, 0), (0, 1)))
    noisepad = jnp.pad(noise, ((0, 0), (0, 1)))
    z128 = jnp.zeros((N, D), jnp.float32)
    z4flat = jnp.zeros((N * 4,), jnp.float32)

    nch = E // _CH
    npass = -(-nch // _NW)
    pad = npass * _NW * _CH - E
    srcT = jnp.pad(src, (0, pad)).reshape(npass, _NW, _CH).transpose(1, 0, 2)
    dstT = jnp.pad(dst, (0, pad)).reshape(npass, _NW, _CH).transpose(1, 0, 2)

    P, A, B = _tc_prep(pospad, noisepad, x, We1[0, :D, :], We1[0, D:2 * D, :],
                       sab)
    h = x
    for l in range(L):
        last = l == L - 1
        pre, relflat = _sc_gather(A, B, P.reshape(N * 4), srcT, dstT, E)
        rel = relflat.reshape(E, 4)
        wc = We1[l, 2 * D, :].reshape(1, D)
        outs = _tc_edge(pre, rel, wc, be1[l].reshape(1, D), We2[l],
                        be2[l].reshape(1, D), Wc1[l], bc1[l].reshape(1, D),
                        Wc2[l].reshape(1, D), bc2[l].reshape(1, 1),
                        write_m=not last)
        if last:
            (wr,) = outs
            apflat = _sc_scatter_pos(wr.reshape(E * 4), dstT, z4flat, N, E)
            ap = _tc_reduce_pos(apflat.reshape(_NW, N * 4)).reshape(N, 4)
            P = _tc_pos_update(P, ap)
        else:
            m, wr = outs
            am2 = _sc_scatter_m(m, dstT, z128, E)
            apflat = _sc_scatter_pos(wr.reshape(E * 4), dstT, z4flat, N, E)
            ap = _tc_reduce_pos(apflat.reshape(_NW, N * 4)).reshape(N, 4)
            h, P, A, B = _tc_node(h, P, am2, ap, Wn1[l, :D, :],
                                  Wn1[l, D:, :], bn1[l].reshape(1, D),
                                  Wn2[l], bn2[l].reshape(1, D),
                                  We1[l + 1, :D, :], We1[l + 1, D:2 * D, :])
    return P[:, :3]
